# Initial kernel scaffold; baseline (speedup 1.0000x reference)
import jax
import jax.numpy as jnp
from jax.experimental import pallas as pl


def kernel(x, table, W):
    raise NotImplementedError("write your pallas kernel here")


# speedup vs baseline: 3.2225x; 3.2225x over previous
import functools

import jax
import jax.numpy as jnp
from jax import lax
from jax.experimental import pallas as pl
from jax.experimental.pallas import tpu as pltpu
from jax.experimental.pallas import tpu_sc as plsc

D = 1024
SEQ = 50
NGRAM = 3
NTERMS = SEQ - (NGRAM - 1)
L = 16
NC, NS = 2, 16
NW = NC * NS
SEQ_PAD = 56


def _sc_encode(x_pad, table):
    B = x_pad.shape[0]
    b_per_w = B // NW
    mesh = plsc.VectorSubcoreMesh(core_axis_name="c", subcore_axis_name="s")

    @functools.partial(
        pl.kernel,
        out_type=jax.ShapeDtypeStruct((B, D), jnp.float32),
        mesh=mesh,
        scratch_types=[
            pltpu.VMEM((b_per_w, SEQ_PAD), jnp.int32),
            pltpu.VMEM((SEQ, D), jnp.float32),
            pltpu.VMEM((D,), jnp.float32),
            pltpu.SemaphoreType.DMA,
        ],
        compiler_params=pltpu.CompilerParams(needs_layout_passes=False),
    )
    def enc_kernel(x_hbm, tab_hbm, out_hbm, idx_v, rows_v, enc_v, sem):
        wid = lax.axis_index("s") * NC + lax.axis_index("c")
        base = wid * b_per_w
        pltpu.sync_copy(x_hbm.at[pl.ds(base, b_per_w)], idx_v)

        lane = lax.iota(jnp.int32, L)
        col_m2 = (lane + (D - 2)) % D
        col_m1 = (lane + (D - 1)) % D

        def per_batch(i, carry):
            pltpu.async_copy(
                tab_hbm.at[idx_v.at[i, pl.ds(0, SEQ)]], rows_v, sem
            ).wait()

            acc = jnp.zeros((L,), jnp.float32)
            for t in range(NTERMS):
                a = plsc.load_gather(
                    rows_v, [jnp.full((L,), t, jnp.int32), col_m2]
                )
                b = plsc.load_gather(
                    rows_v, [jnp.full((L,), t + 1, jnp.int32), col_m1]
                )
                c0 = rows_v[t + 2, pl.ds(0, L)]
                acc = acc + a * b * c0
            enc_v[pl.ds(0, L)] = jnp.where(acc > 0, 1.0, -1.0)

            def chunk_body(cc, carry2):
                d0 = cc * L
                acc2 = jnp.zeros((L,), jnp.float32)
                for t in range(NTERMS):
                    a = rows_v[t, pl.ds(d0 - 2, L)]
                    b = rows_v[t + 1, pl.ds(d0 - 1, L)]
                    c2 = rows_v[t + 2, pl.ds(d0, L)]
                    acc2 = acc2 + a * b * c2
                enc_v[pl.ds(d0, L)] = jnp.where(acc2 > 0, 1.0, -1.0)
                return carry2

            lax.fori_loop(1, D // L, chunk_body, 0)
            pltpu.sync_copy(enc_v, out_hbm.at[base + i])
            return carry

        lax.fori_loop(0, b_per_w, per_batch, 0)

    return enc_kernel(x_pad, table)


def _classify(enc, W):
    B = enc.shape[0]
    NCLS = W.shape[0]

    def mm_kernel(enc_ref, w_ref, out_ref):
        out_ref[...] = lax.dot_general(
            enc_ref[...], w_ref[...],
            (((1,), (1,)), ((), ())),
            preferred_element_type=jnp.float32,
        )

    return pl.pallas_call(
        mm_kernel,
        out_shape=jax.ShapeDtypeStruct((B, NCLS), jnp.float32),
    )(enc, W)


def kernel(x, table, W):
    x_pad = jnp.pad(x.astype(jnp.int32), ((0, 0), (0, SEQ_PAD - SEQ)))
    enc = _sc_encode(x_pad, table)
    return _classify(enc, W)


# speedup vs baseline: 4.2994x; 1.3342x over previous
import functools

import jax
import jax.numpy as jnp
from jax import lax
from jax.experimental import pallas as pl
from jax.experimental.pallas import tpu as pltpu
from jax.experimental.pallas import tpu_sc as plsc

D = 1024
SEQ = 50
NGRAM = 3
NTERMS = SEQ - (NGRAM - 1)
L = 16
NC, NS = 2, 16
NW = NC * NS
SEQ_PAD = 56


def _sc_encode(x_pad, table):
    B = x_pad.shape[0]
    b_per_w = B // NW
    mesh = plsc.VectorSubcoreMesh(core_axis_name="c", subcore_axis_name="s")

    @functools.partial(
        pl.kernel,
        out_type=jax.ShapeDtypeStruct((B, D), jnp.float32),
        mesh=mesh,
        scratch_types=[
            pltpu.VMEM((b_per_w, SEQ_PAD), jnp.int32),
            pltpu.VMEM((SEQ, D), jnp.float32),
            pltpu.VMEM((SEQ, D), jnp.float32),
            pltpu.VMEM((D,), jnp.float32),
            pltpu.SemaphoreType.DMA,
            pltpu.SemaphoreType.DMA,
        ],
        compiler_params=pltpu.CompilerParams(needs_layout_passes=False),
    )
    def enc_kernel(x_hbm, tab_hbm, out_hbm, idx_v, rows0, rows1, enc_v,
                   sem0, sem1):
        wid = lax.axis_index("s") * NC + lax.axis_index("c")
        base = wid * b_per_w
        pltpu.sync_copy(x_hbm.at[pl.ds(base, b_per_w)], idx_v)

        lane = lax.iota(jnp.int32, L)
        col_m2 = (lane + (D - 2)) % D
        col_m1 = (lane + (D - 1)) % D

        def gather_start(b, rows, sem):
            pltpu.async_copy(
                tab_hbm.at[idx_v.at[b, pl.ds(0, SEQ)]], rows, sem
            )

        def gather_wait(rows, sem):
            pltpu.make_async_copy(
                tab_hbm.at[idx_v.at[0, pl.ds(0, SEQ)]], rows, sem
            ).wait()

        def compute_enc(rows_v, b):
            acc = jnp.zeros((L,), jnp.float32)
            for t in range(NTERMS):
                a = plsc.load_gather(
                    rows_v, [jnp.full((L,), t, jnp.int32), col_m2]
                )
                b_ = plsc.load_gather(
                    rows_v, [jnp.full((L,), t + 1, jnp.int32), col_m1]
                )
                c0 = rows_v[t + 2, pl.ds(0, L)]
                acc = acc + a * b_ * c0
            enc_v[pl.ds(0, L)] = jnp.where(acc > 0, 1.0, -1.0)

            def chunk_body(cc, carry2):
                d0 = cc * L
                acc2 = jnp.zeros((L,), jnp.float32)
                for t in range(NTERMS):
                    a = rows_v[t, pl.ds(d0 - 2, L)]
                    b_ = rows_v[t + 1, pl.ds(d0 - 1, L)]
                    c2 = rows_v[t + 2, pl.ds(d0, L)]
                    acc2 = acc2 + a * b_ * c2
                enc_v[pl.ds(d0, L)] = jnp.where(acc2 > 0, 1.0, -1.0)
                return carry2

            lax.fori_loop(1, D // L, chunk_body, 0)
            pltpu.sync_copy(enc_v, out_hbm.at[base + b])

        gather_start(0, rows0, sem0)

        def pair_body(i, carry):
            b0 = 2 * i
            gather_wait(rows0, sem0)
            gather_start(b0 + 1, rows1, sem1)
            compute_enc(rows0, b0)
            gather_wait(rows1, sem1)

            @pl.when(i < b_per_w // 2 - 1)
            def _():
                gather_start(b0 + 2, rows0, sem0)

            compute_enc(rows1, b0 + 1)
            return carry

        lax.fori_loop(0, b_per_w // 2, pair_body, 0)

    return enc_kernel(x_pad, table)


def _classify(enc, W):
    B = enc.shape[0]
    NCLS = W.shape[0]

    def mm_kernel(enc_ref, w_ref, out_ref):
        out_ref[...] = lax.dot_general(
            enc_ref[...], w_ref[...],
            (((1,), (1,)), ((), ())),
            preferred_element_type=jnp.float32,
        )

    return pl.pallas_call(
        mm_kernel,
        out_shape=jax.ShapeDtypeStruct((B, NCLS), jnp.float32),
    )(enc, W)


def kernel(x, table, W):
    x_pad = jnp.pad(x.astype(jnp.int32), ((0, 0), (0, SEQ_PAD - SEQ)))
    enc = _sc_encode(x_pad, table)
    return _classify(enc, W)
